# baseline (speedup 1.0000x reference)
import functools

import jax
import jax.numpy as jnp
from jax import lax
from jax.experimental import pallas as pl
from jax.experimental.pallas import tpu as pltpu
from jax.experimental.pallas import tpu_sc as plsc

_NC, _NS, _L = 2, 16, 16
_NW = _NC * _NS


def _sc_build_counts(edge_index, n):
    e = edge_index.shape[1]
    rpt = n // _NW
    assert rpt == _L and e % _L == 0 and n % _L == 0
    mesh = plsc.VectorSubcoreMesh(core_axis_name="c", subcore_axis_name="s")

    @functools.partial(
        pl.kernel, mesh=mesh,
        compiler_params=pltpu.CompilerParams(needs_layout_passes=False),
        out_type=jax.ShapeDtypeStruct((_NW, rpt * n), jnp.float32),
        scratch_types=[
            pltpu.VMEM((e,), jnp.int32),
            pltpu.VMEM((e,), jnp.int32),
            pltpu.VMEM((rpt * n,), jnp.float32),
        ],
    )
    def build(edge_hbm, out_hbm, src_v, dst_v, strip_v):
        wid = lax.axis_index("s") * _NC + lax.axis_index("c")
        lo = wid * rpt
        pltpu.sync_copy(edge_hbm.at[0], src_v)
        pltpu.sync_copy(edge_hbm.at[1], dst_v)

        zeros = jnp.zeros((_L,), jnp.float32)

        @plsc.parallel_loop(0, rpt * n // _L, unroll=8)
        def zchunk(k):
            strip_v[pl.ds(k * _L, _L)] = zeros

        ones = jnp.ones((_L,), jnp.float32)

        @plsc.parallel_loop(0, e // _L, unroll=8)
        def chunk(j):
            s = src_v[pl.ds(j * _L, _L)]
            d = dst_v[pl.ds(j * _L, _L)]
            r = d - lo
            m = (r >= 0) & (r < rpt)
            rc = jnp.clip(r, 0, rpt - 1)
            plsc.addupdate_scatter(strip_v, [rc * n + s], ones, mask=m)

        ri = lax.broadcasted_iota(jnp.int32, (_L,), 0)
        plsc.addupdate_scatter(strip_v, [ri * n + lo + ri], ones)

        pltpu.sync_copy(strip_v, out_hbm.at[wid])

    return build(edge_index).reshape(n, n)


_SQRT_2PI = 2.5066282746310002


def _log_sigmoid(x):
    return jnp.minimum(x, 0.0) - jnp.log1p(jnp.exp(-jnp.abs(x)))


def _tc_body(x1_ref, x2_ref, cnt_ref, w1_ref, b1_ref, w2_ref, b2_ref,
             out_ref, ahat_ref):
    b = pl.program_id(0)
    n = cnt_ref.shape[0]

    @pl.when(b == 0)
    def _init():
        cnt = cnt_ref[...]
        deg = jnp.sum(cnt, axis=1)
        dinv = lax.rsqrt(jnp.maximum(deg, 1.0))
        ahat_ref[...] = (cnt * dinv[:, None] * dinv[None, :]
                         ).astype(jnp.bfloat16)
        out_ref[0] = 0.0
        out_ref[1] = 0.0
        out_ref[2] = 0.0

    A = ahat_ref[...]
    W1 = w1_ref[...]
    W2 = w2_ref[...]
    b1 = b1_ref[...]
    b2 = b2_ref[...]

    def enc(x):
        xw = jnp.dot(x.astype(jnp.bfloat16), W1,
                     preferred_element_type=jnp.float32)
        h = jnp.maximum(
            jnp.dot(A, xw.astype(jnp.bfloat16),
                    preferred_element_type=jnp.float32) + b1, 0.0)
        hw = jnp.dot(h.astype(jnp.bfloat16), W2,
                     preferred_element_type=jnp.float32)
        return jnp.maximum(
            jnp.dot(A, hw.astype(jnp.bfloat16),
                    preferred_element_type=jnp.float32) + b2, 0.0)

    ln2 = jnp.float32(0.6931471805599453)
    nn = jnp.float32(n)

    p1 = jnp.float32(0.0)
    p2 = jnp.float32(0.0)
    p3 = jnp.float32(0.0)
    for i in range(x1_ref.shape[0]):
        H1 = enc(x1_ref[i])
        H2 = enc(x2_ref[i])

        S = lax.dot_general(H1.astype(jnp.bfloat16), H2.astype(jnp.bfloat16),
                            (((1,), (1,)), ((), ())),
                            preferred_element_type=jnp.float32)
        h1sum = jnp.sum(H1, axis=0)
        colsum = jnp.sum(H2 * h1sum, axis=1)
        p1 += jnp.sum(colsum / jnp.clip(colsum, 1e-6, None))

        Mu = h1sum / nn
        Su = jnp.sqrt(jnp.sum((H1 - Mu) ** 2, axis=0) / (nn - 1.0)) + 1e-12
        Mv = jnp.sum(H2, axis=0) / nn
        Sv = jnp.sqrt(jnp.sum((H2 - Mv) ** 2, axis=0) / (nn - 1.0)) + 1e-12

        def npdf(loc, scale):
            z = (H2 - loc) * (1.0 / scale)
            return jnp.exp(-0.5 * z * z) * (1.0 / (scale * _SQRT_2PI))

        pu = npdf(Mu, Su)
        pv = npdf(Mv, Sv)
        pmat = pu / (pu + pv + 2e-12)
        hardf = (jnp.sum(pmat, axis=1) * (1.0 / pmat.shape[1]) > 0.6
                 ).astype(jnp.float32)
        nhard = jnp.sum(hardf)

        dvec = jnp.sum(H1 * H2, axis=1)
        p2 += jnp.sum(_log_sigmoid(dvec)) - jnp.float32(n * n - n) * ln2
        lsn = _log_sigmoid(-S)
        rowsum = jnp.sum(lsn, axis=1)
        lsdn = _log_sigmoid(-dvec)
        p3 += (jnp.sum(rowsum)
               - jnp.sum(hardf * rowsum)
               - jnp.sum(lsdn)
               + jnp.sum(hardf * lsdn)
               - (nn + (nn - 1.0) * nhard) * ln2)

    out_ref[0] += p1
    out_ref[1] += p2
    out_ref[2] += p3

    @pl.when(b == pl.num_programs(0) - 1)
    def _fin():
        nb = jnp.float32(pl.num_programs(0) * x1_ref.shape[0])
        total = nb * nn * nn
        node_loss = -jnp.log(jnp.clip(out_ref[0] / total, 1e-6, None))
        pos_loss = out_ref[1] / total
        neg_loss = out_ref[2] / (nb * nn)
        out_ref[3] = node_loss + 0.1 * (-(0.5 * pos_loss + 0.5 * neg_loss))


def kernel(x1, x2, edge_index, A1, A2, W1, b1, W2, b2):
    del A1, A2
    bsz, n, f = x1.shape
    h_mid = W1.shape[1]
    h_out = W2.shape[1]

    cnt = _sc_build_counts(edge_index, n)

    bpb = 8
    partials = pl.pallas_call(
        _tc_body,
        grid=(bsz // bpb,),
        in_specs=[
            pl.BlockSpec((bpb, n, f), lambda b: (b, 0, 0)),
            pl.BlockSpec((bpb, n, f), lambda b: (b, 0, 0)),
            pl.BlockSpec((n, n), lambda b: (0, 0)),
            pl.BlockSpec((f, h_mid), lambda b: (0, 0)),
            pl.BlockSpec((1, h_mid), lambda b: (0, 0)),
            pl.BlockSpec((h_mid, h_out), lambda b: (0, 0)),
            pl.BlockSpec((1, h_out), lambda b: (0, 0)),
        ],
        out_specs=pl.BlockSpec(memory_space=pltpu.SMEM),
        out_shape=jax.ShapeDtypeStruct((4,), jnp.float32),
        scratch_shapes=[pltpu.VMEM((n, n), jnp.bfloat16)],
    )(x1, x2, cnt, W1.astype(jnp.bfloat16), b1.reshape(1, h_mid),
      W2.astype(jnp.bfloat16), b2.reshape(1, h_out))

    return partials[3]


# speedup vs baseline: 1.0016x; 1.0016x over previous
import functools

import jax
import jax.numpy as jnp
from jax import lax
from jax.experimental import pallas as pl
from jax.experimental.pallas import tpu as pltpu
from jax.experimental.pallas import tpu_sc as plsc

_NC, _NS, _L = 2, 16, 16
_NW = _NC * _NS


def _sc_build_counts(edge_index, n):
    e = edge_index.shape[1]
    rpt = n // _NW
    assert rpt == _L and e % _L == 0 and n % _L == 0
    mesh = plsc.VectorSubcoreMesh(core_axis_name="c", subcore_axis_name="s")

    @functools.partial(
        pl.kernel, mesh=mesh,
        compiler_params=pltpu.CompilerParams(needs_layout_passes=False),
        out_type=jax.ShapeDtypeStruct((_NW, rpt * n), jnp.float32),
        scratch_types=[
            pltpu.VMEM((e,), jnp.int32),
            pltpu.VMEM((e,), jnp.int32),
            pltpu.VMEM((rpt * n,), jnp.float32),
        ],
    )
    def build(edge_hbm, out_hbm, src_v, dst_v, strip_v):
        wid = lax.axis_index("s") * _NC + lax.axis_index("c")
        lo = wid * rpt
        pltpu.sync_copy(edge_hbm.at[0], src_v)
        pltpu.sync_copy(edge_hbm.at[1], dst_v)

        zeros = jnp.zeros((_L,), jnp.float32)

        @plsc.parallel_loop(0, rpt * n // _L, unroll=8)
        def zchunk(k):
            strip_v[pl.ds(k * _L, _L)] = zeros

        ones = jnp.ones((_L,), jnp.float32)

        @plsc.parallel_loop(0, e // _L, unroll=8)
        def chunk(j):
            s = src_v[pl.ds(j * _L, _L)]
            d = dst_v[pl.ds(j * _L, _L)]
            r = d - lo
            m = (r >= 0) & (r < rpt)
            rc = jnp.clip(r, 0, rpt - 1)
            plsc.addupdate_scatter(strip_v, [rc * n + s], ones, mask=m)

        ri = lax.broadcasted_iota(jnp.int32, (_L,), 0)
        plsc.addupdate_scatter(strip_v, [ri * n + lo + ri], ones)

        pltpu.sync_copy(strip_v, out_hbm.at[wid])

    return build(edge_index).reshape(n, n)


_SQRT_2PI = 2.5066282746310002


def _log_sigmoid(x):
    return jnp.minimum(x, 0.0) - jnp.log1p(jnp.exp(-jnp.abs(x)))


def _tc_body(x1_ref, x2_ref, cnt_ref, w1_ref, b1_ref, w2_ref, b2_ref,
             out_ref, loss_ref, ahat_ref):
    b = pl.program_id(0)
    n = ahat_ref.shape[0]

    @pl.when(b == 0)
    def _init():
        cnt = cnt_ref[...]
        deg = jnp.sum(cnt, axis=1)
        dinv = lax.rsqrt(jnp.maximum(deg, 1.0))
        ahat_ref[...] = (cnt * dinv[:, None] * dinv[None, :]
                         ).astype(jnp.bfloat16)
        out_ref[0] = 0.0
        out_ref[1] = 0.0
        out_ref[2] = 0.0

    A = ahat_ref[...]
    W1 = w1_ref[...]
    W2 = w2_ref[...]
    b1 = b1_ref[...]
    b2 = b2_ref[...]

    def enc(x):
        xw = jnp.dot(x.astype(jnp.bfloat16), W1,
                     preferred_element_type=jnp.float32)
        h = jnp.maximum(
            jnp.dot(A, xw.astype(jnp.bfloat16),
                    preferred_element_type=jnp.float32) + b1, 0.0)
        hw = jnp.dot(h.astype(jnp.bfloat16), W2,
                     preferred_element_type=jnp.float32)
        return jnp.maximum(
            jnp.dot(A, hw.astype(jnp.bfloat16),
                    preferred_element_type=jnp.float32) + b2, 0.0)

    ln2 = jnp.float32(0.6931471805599453)
    nn = jnp.float32(n)

    p1 = jnp.float32(0.0)
    p2 = jnp.float32(0.0)
    p3 = jnp.float32(0.0)
    for i in range(x1_ref.shape[0]):
        H1 = enc(x1_ref[i])
        H2 = enc(x2_ref[i])

        S = lax.dot_general(H1.astype(jnp.bfloat16), H2.astype(jnp.bfloat16),
                            (((1,), (1,)), ((), ())),
                            preferred_element_type=jnp.float32)
        h1sum = jnp.sum(H1, axis=0)
        colsum = jnp.sum(H2 * h1sum, axis=1)
        p1 += jnp.sum(colsum / jnp.clip(colsum, 1e-6, None))

        Mu = h1sum / nn
        Su = jnp.sqrt(jnp.sum((H1 - Mu) ** 2, axis=0) / (nn - 1.0)) + 1e-12
        Mv = jnp.sum(H2, axis=0) / nn
        Sv = jnp.sqrt(jnp.sum((H2 - Mv) ** 2, axis=0) / (nn - 1.0)) + 1e-12

        def npdf(loc, scale):
            z = (H2 - loc) * (1.0 / scale)
            return jnp.exp(-0.5 * z * z) * (1.0 / (scale * _SQRT_2PI))

        pu = npdf(Mu, Su)
        pv = npdf(Mv, Sv)
        pmat = pu / (pu + pv + 2e-12)
        wv = 1.0 - (jnp.sum(pmat, axis=1) * (1.0 / pmat.shape[1]) > 0.6
                    ).astype(jnp.float32)
        sumw = jnp.sum(wv)

        dvec = jnp.sum(H1 * H2, axis=1)
        p2 += jnp.sum(_log_sigmoid(dvec)) - jnp.float32(n * n - n) * ln2
        lsn = _log_sigmoid(-S)
        lsdn = _log_sigmoid(-dvec)
        p3 += (jnp.sum(wv[:, None] * lsn)
               - jnp.sum(wv * lsdn)
               - (nn * nn - (nn - 1.0) * sumw) * ln2)

    out_ref[0] += p1
    out_ref[1] += p2
    out_ref[2] += p3

    @pl.when(b == pl.num_programs(0) - 1)
    def _fin():
        nb = jnp.float32(pl.num_programs(0) * x1_ref.shape[0])
        total = nb * nn * nn
        node_loss = -jnp.log(jnp.clip(out_ref[0] / total, 1e-6, None))
        pos_loss = out_ref[1] / total
        neg_loss = out_ref[2] / (nb * nn)
        loss_ref[0] = node_loss + 0.1 * (-(0.5 * pos_loss + 0.5 * neg_loss))


def kernel(x1, x2, edge_index, A1, A2, W1, b1, W2, b2):
    del A1, A2
    bsz, n, f = x1.shape
    h_mid = W1.shape[1]
    h_out = W2.shape[1]

    cnt = _sc_build_counts(edge_index, n)

    bpb = 8
    _, loss = pl.pallas_call(
        _tc_body,
        grid=(bsz // bpb,),
        in_specs=[
            pl.BlockSpec((bpb, n, f), lambda b: (b, 0, 0)),
            pl.BlockSpec((bpb, n, f), lambda b: (b, 0, 0)),
            pl.BlockSpec((n, n), lambda b: (0, 0)),
            pl.BlockSpec((f, h_mid), lambda b: (0, 0)),
            pl.BlockSpec((1, h_mid), lambda b: (0, 0)),
            pl.BlockSpec((h_mid, h_out), lambda b: (0, 0)),
            pl.BlockSpec((1, h_out), lambda b: (0, 0)),
        ],
        out_specs=[pl.BlockSpec(memory_space=pltpu.SMEM),
                   pl.BlockSpec(memory_space=pltpu.SMEM)],
        out_shape=[jax.ShapeDtypeStruct((3,), jnp.float32),
                   jax.ShapeDtypeStruct((1,), jnp.float32)],
        scratch_shapes=[pltpu.VMEM((n, n), jnp.bfloat16)],
    )(x1, x2, cnt, W1.astype(jnp.bfloat16), b1.reshape(1, h_mid),
      W2.astype(jnp.bfloat16), b2.reshape(1, h_out))

    return loss.reshape(())


# speedup vs baseline: 1.0111x; 1.0095x over previous
import functools

import jax
import jax.numpy as jnp
from jax import lax
from jax.experimental import pallas as pl
from jax.experimental.pallas import tpu as pltpu
from jax.experimental.pallas import tpu_sc as plsc

_NC, _NS, _L = 2, 16, 16
_NW = _NC * _NS


def _sc_build_counts(edge_index, n):
    e = edge_index.shape[1]
    rpt = n // _NW
    assert rpt == _L and e % _L == 0 and n % _L == 0
    mesh = plsc.VectorSubcoreMesh(core_axis_name="c", subcore_axis_name="s")

    @functools.partial(
        pl.kernel, mesh=mesh,
        compiler_params=pltpu.CompilerParams(needs_layout_passes=False),
        out_type=jax.ShapeDtypeStruct((_NW, rpt * n), jnp.float32),
        scratch_types=[
            pltpu.VMEM((e,), jnp.int32),
            pltpu.VMEM((e,), jnp.int32),
            pltpu.VMEM((rpt * n,), jnp.float32),
        ],
    )
    def build(edge_hbm, out_hbm, src_v, dst_v, strip_v):
        wid = lax.axis_index("s") * _NC + lax.axis_index("c")
        lo = wid * rpt
        pltpu.sync_copy(edge_hbm.at[0], src_v)
        pltpu.sync_copy(edge_hbm.at[1], dst_v)

        zeros = jnp.zeros((_L,), jnp.float32)

        @plsc.parallel_loop(0, rpt * n // _L, unroll=8)
        def zchunk(k):
            strip_v[pl.ds(k * _L, _L)] = zeros

        ones = jnp.ones((_L,), jnp.float32)

        @plsc.parallel_loop(0, e // _L, unroll=8)
        def chunk(j):
            s = src_v[pl.ds(j * _L, _L)]
            d = dst_v[pl.ds(j * _L, _L)]
            r = d - lo
            m = (r >= 0) & (r < rpt)
            rc = jnp.clip(r, 0, rpt - 1)
            plsc.addupdate_scatter(strip_v, [rc * n + s], ones, mask=m)

        ri = lax.broadcasted_iota(jnp.int32, (_L,), 0)
        plsc.addupdate_scatter(strip_v, [ri * n + lo + ri], ones)

        pltpu.sync_copy(strip_v, out_hbm.at[wid])

    return build(edge_index).reshape(n, n)


_SQRT_2PI = 2.5066282746310002


def _log_sigmoid(x):
    return jnp.minimum(x, 0.0) - jnp.log1p(jnp.exp(-jnp.abs(x)))


def _tc_body(x1_ref, x2_ref, cnt_ref, w1_ref, b1_ref, w2_ref, b2_ref,
             out_ref, loss_ref, ahat_ref):
    b = pl.program_id(0)
    n = ahat_ref.shape[0]

    @pl.when(b == 0)
    def _init():
        cnt = cnt_ref[...]
        deg = jnp.sum(cnt, axis=1)
        dinv = lax.rsqrt(jnp.maximum(deg, 1.0))
        ahat_ref[...] = cnt * dinv[:, None] * dinv[None, :]
        out_ref[0] = 0.0
        out_ref[1] = 0.0
        out_ref[2] = 0.0

    A = ahat_ref[...]
    W1 = w1_ref[...]
    W2 = w2_ref[...]
    b1 = b1_ref[...]
    b2 = b2_ref[...]

    def enc(x):
        xw = jnp.dot(x, W1, preferred_element_type=jnp.float32)
        h = jnp.maximum(jnp.dot(A, xw, preferred_element_type=jnp.float32) + b1, 0.0)
        hw = jnp.dot(h, W2, preferred_element_type=jnp.float32)
        return jnp.maximum(jnp.dot(A, hw, preferred_element_type=jnp.float32) + b2, 0.0)

    ln2 = jnp.float32(0.6931471805599453)
    nn = jnp.float32(n)

    p1 = jnp.float32(0.0)
    p2 = jnp.float32(0.0)
    p3 = jnp.float32(0.0)
    for i in range(x1_ref.shape[0]):
        H1 = enc(x1_ref[i])
        H2 = enc(x2_ref[i])

        S = lax.dot_general(H1, H2, (((1,), (1,)), ((), ())),
                            preferred_element_type=jnp.float32)
        h1sum = jnp.sum(H1, axis=0)
        colsum = jnp.sum(H2 * h1sum, axis=1)
        p1 += jnp.sum(colsum / jnp.clip(colsum, 1e-6, None))

        Mu = h1sum / nn
        Su = jnp.sqrt(jnp.sum((H1 - Mu) ** 2, axis=0) / (nn - 1.0)) + 1e-12
        Mv = jnp.sum(H2, axis=0) / nn
        Sv = jnp.sqrt(jnp.sum((H2 - Mv) ** 2, axis=0) / (nn - 1.0)) + 1e-12

        def npdf(loc, scale):
            z = (H2 - loc) * (1.0 / scale)
            return jnp.exp(-0.5 * z * z) * (1.0 / (scale * _SQRT_2PI))

        pu = npdf(Mu, Su)
        pv = npdf(Mv, Sv)
        pmat = pu / (pu + pv + 2e-12)
        hardf = (jnp.sum(pmat, axis=1) * (1.0 / pmat.shape[1]) > 0.6
                 ).astype(jnp.float32)
        nhard = jnp.sum(hardf)

        dvec = jnp.sum(H1 * H2, axis=1)
        p2 += jnp.sum(_log_sigmoid(dvec)) - jnp.float32(n * n - n) * ln2
        lsn = _log_sigmoid(-S)
        rowsum = jnp.sum(lsn, axis=1)
        lsdn = _log_sigmoid(-dvec)
        p3 += (jnp.sum(rowsum)
               - jnp.sum(hardf * rowsum)
               - jnp.sum(lsdn)
               + jnp.sum(hardf * lsdn)
               - (nn + (nn - 1.0) * nhard) * ln2)

    out_ref[0] += p1
    out_ref[1] += p2
    out_ref[2] += p3

    @pl.when(b == pl.num_programs(0) - 1)
    def _fin():
        nb = jnp.float32(pl.num_programs(0) * x1_ref.shape[0])
        total = nb * nn * nn
        node_loss = -jnp.log(jnp.clip(out_ref[0] / total, 1e-6, None))
        pos_loss = out_ref[1] / total
        neg_loss = out_ref[2] / (nb * nn)
        loss_ref[0] = node_loss + 0.1 * (-(0.5 * pos_loss + 0.5 * neg_loss))


def kernel(x1, x2, edge_index, A1, A2, W1, b1, W2, b2):
    del A1, A2
    bsz, n, f = x1.shape
    h_mid = W1.shape[1]
    h_out = W2.shape[1]

    cnt = _sc_build_counts(edge_index, n)

    bpb = 8
    _, loss = pl.pallas_call(
        _tc_body,
        grid=(bsz // bpb,),
        in_specs=[
            pl.BlockSpec((bpb, n, f), lambda b: (b, 0, 0)),
            pl.BlockSpec((bpb, n, f), lambda b: (b, 0, 0)),
            pl.BlockSpec((n, n), lambda b: (0, 0)),
            pl.BlockSpec((f, h_mid), lambda b: (0, 0)),
            pl.BlockSpec((1, h_mid), lambda b: (0, 0)),
            pl.BlockSpec((h_mid, h_out), lambda b: (0, 0)),
            pl.BlockSpec((1, h_out), lambda b: (0, 0)),
        ],
        out_specs=[pl.BlockSpec(memory_space=pltpu.SMEM),
                   pl.BlockSpec(memory_space=pltpu.SMEM)],
        out_shape=[jax.ShapeDtypeStruct((3,), jnp.float32),
                   jax.ShapeDtypeStruct((1,), jnp.float32)],
        scratch_shapes=[pltpu.VMEM((n, n), jnp.float32)],
    )(x1, x2, cnt, W1, b1.reshape(1, h_mid), W2, b2.reshape(1, h_out))

    return loss.reshape(())
